# Initial kernel scaffold; baseline (speedup 1.0000x reference)
import jax
import jax.numpy as jnp
from jax.experimental import pallas as pl


def kernel(x, edge_index, edge_attr, Wl1, bl1, Wr1, br1, Wl2, bl2, Wr2, br2, Wl3, bl3, Wr3, br3, g1, be1, g2, be2):
    raise NotImplementedError("write your pallas kernel here")


# speedup vs baseline: 3.6566x; 3.6566x over previous
import jax
import jax.numpy as jnp
from jax import lax
from jax.experimental import pallas as pl
from jax.experimental.pallas import tpu as pltpu
from jax.experimental.pallas import tpu_sc as plsc

_N = 10000
_E = 320000
_D = 128

_NC = 2
_NS = 16
_NW = _NC * _NS
_EPT = _E // _NW
_CHUNK = 80
_NCHUNK = _EPT // _CHUNK
_RPT = _N // _NS
_ZROWS = 25

_sc_mesh = plsc.VectorSubcoreMesh(core_axis_name="c", subcore_axis_name="s")


def _segsum_body(h_hbm, src_hbm, dst_hbm, ew_hbm, out_hbm,
                 agg_sh, zrow_v, src_v, dst_v, ew_v, rows_v, sem):
    c = lax.axis_index("c")
    s = lax.axis_index("s")
    wid = c * _NS + s

    zero16 = jnp.zeros((16,), jnp.float32)
    for r in range(_ZROWS):
        for dc in range(_D // 16):
            zrow_v[r, pl.ds(dc * 16, 16)] = zero16

    def _zero(i, carry):
        pltpu.sync_copy(zrow_v, agg_sh.at[pl.ds(s * _RPT + i * _ZROWS, _ZROWS)])
        return carry
    lax.fori_loop(0, _RPT // _ZROWS, _zero, 0)
    plsc.subcore_barrier()

    def _edges(ch, carry):
        base = wid * _EPT + ch * _CHUNK
        pltpu.sync_copy(src_hbm.at[pl.ds(base, _CHUNK)], src_v)
        pltpu.sync_copy(dst_hbm.at[pl.ds(base, _CHUNK)], dst_v)
        pltpu.sync_copy(ew_hbm.at[pl.ds(base, _CHUNK)], ew_v)
        pltpu.async_copy(h_hbm.at[src_v], rows_v, sem).wait()

        def _scale(m, c2):
            ew_vec = ew_v[pl.ds(pl.multiple_of(m * 16, 16), 16)]
            for j in range(16):
                w = jnp.broadcast_to(ew_vec[j], (16,))
                e = m * 16 + j
                for dc in range(_D // 16):
                    sl = pl.ds(dc * 16, 16)
                    rows_v[e, sl] = rows_v[e, sl] * w
            return c2
        lax.fori_loop(0, _CHUNK // 16, _scale, 0)

        pltpu.sync_copy(rows_v, agg_sh.at[dst_v], add=True)
        return carry
    lax.fori_loop(0, _NCHUNK, _edges, 0)
    plsc.subcore_barrier()
    _writeout(agg_sh, out_hbm, c, s)


def _writeout(shared, out_hbm, c, s):
    start = pl.multiple_of(s * 632, 8)
    off = pl.multiple_of(c * _N + s * 632, 8)

    @pl.when(s < _NS - 1)
    def _():
        pltpu.sync_copy(shared.at[pl.ds(start, 632)],
                        out_hbm.at[pl.ds(off, 632)])

    @pl.when(s == _NS - 1)
    def _():
        pltpu.sync_copy(shared.at[pl.ds(start, 520)],
                        out_hbm.at[pl.ds(off, 520)])


_segsum = pl.kernel(
    _segsum_body,
    mesh=_sc_mesh,
    out_type=jax.ShapeDtypeStruct((2 * _N, _D), jnp.float32),
    scratch_types=[
        pltpu.VMEM_SHARED((_N, _D), jnp.float32),
        pltpu.VMEM((_ZROWS, _D), jnp.float32),
        pltpu.VMEM((_CHUNK,), jnp.int32),
        pltpu.VMEM((_CHUNK,), jnp.int32),
        pltpu.VMEM((_CHUNK,), jnp.float32),
        pltpu.VMEM((_CHUNK, _D), jnp.float32),
        pltpu.SemaphoreType.DMA,
    ],
)


def _count_body(dst_hbm, out_hbm, cnt_sh, zrow_v, dst_v, ones_v):
    c = lax.axis_index("c")
    s = lax.axis_index("s")
    wid = c * _NS + s

    zero16 = jnp.zeros((16,), jnp.float32)
    one16 = jnp.ones((16,), jnp.float32)
    for r in range(_ZROWS):
        for dc in range(_D // 16):
            zrow_v[r, pl.ds(dc * 16, 16)] = zero16

    def _ones(r, carry):
        for dc in range(_D // 16):
            ones_v[r, pl.ds(dc * 16, 16)] = one16
        return carry
    lax.fori_loop(0, _CHUNK, _ones, 0)

    def _zero(i, carry):
        pltpu.sync_copy(zrow_v, cnt_sh.at[pl.ds(s * _RPT + i * _ZROWS, _ZROWS)])
        return carry
    lax.fori_loop(0, _RPT // _ZROWS, _zero, 0)
    plsc.subcore_barrier()

    def _edges(ch, carry):
        base = wid * _EPT + ch * _CHUNK
        pltpu.sync_copy(dst_hbm.at[pl.ds(base, _CHUNK)], dst_v)
        pltpu.sync_copy(ones_v, cnt_sh.at[dst_v], add=True)
        return carry
    lax.fori_loop(0, _NCHUNK, _edges, 0)
    plsc.subcore_barrier()
    _writeout(cnt_sh, out_hbm, c, s)


_count = pl.kernel(
    _count_body,
    mesh=_sc_mesh,
    out_type=jax.ShapeDtypeStruct((2 * _N, _D), jnp.float32),
    scratch_types=[
        pltpu.VMEM_SHARED((_N, _D), jnp.float32),
        pltpu.VMEM((_ZROWS, _D), jnp.float32),
        pltpu.VMEM((_CHUNK,), jnp.int32),
        pltpu.VMEM((_CHUNK, _D), jnp.float32),
    ],
)


_BN = 1000


def _linear_stats_body(parts_ref, cnt_ref, h_ref, wl_ref, wr_ref, b_ref,
                       z_ref, sum_ref, sq_ref):
    i = pl.program_id(0)
    cnt = cnt_ref[0] + cnt_ref[1]
    a = (parts_ref[0] + parts_ref[1]) / jnp.maximum(cnt, 1.0)
    z = (jnp.dot(a, wl_ref[...], preferred_element_type=jnp.float32)
         + jnp.dot(h_ref[...], wr_ref[...], preferred_element_type=jnp.float32)
         + b_ref[...])
    z_ref[...] = z

    @pl.when(i == 0)
    def _():
        sum_ref[...] = jnp.sum(z, 0, keepdims=True)
        sq_ref[...] = jnp.sum(z * z, 0, keepdims=True)

    @pl.when(i != 0)
    def _():
        sum_ref[...] += jnp.sum(z, 0, keepdims=True)
        sq_ref[...] += jnp.sum(z * z, 0, keepdims=True)


def _linear_plain_body(parts_ref, cnt_ref, h_ref, wl_ref, wr_ref, b_ref,
                       z_ref):
    cnt = cnt_ref[0] + cnt_ref[1]
    a = (parts_ref[0] + parts_ref[1]) / jnp.maximum(cnt, 1.0)
    z_ref[...] = (jnp.dot(a, wl_ref[...], preferred_element_type=jnp.float32)
                  + jnp.dot(h_ref[...], wr_ref[...],
                            preferred_element_type=jnp.float32)
                  + b_ref[...])


_lin_in_specs = [
    pl.BlockSpec((2, _BN, _D), lambda i: (0, i, 0)),
    pl.BlockSpec((2, _BN, 1), lambda i: (0, i, 0)),
    pl.BlockSpec((_BN, _D), lambda i: (i, 0)),
    pl.BlockSpec((_D, _D), lambda i: (0, 0)),
    pl.BlockSpec((_D, _D), lambda i: (0, 0)),
    pl.BlockSpec((1, _D), lambda i: (0, 0)),
]

_linear_stats = pl.pallas_call(
    _linear_stats_body,
    grid=(_N // _BN,),
    in_specs=_lin_in_specs,
    out_specs=[
        pl.BlockSpec((_BN, _D), lambda i: (i, 0)),
        pl.BlockSpec((1, _D), lambda i: (0, 0)),
        pl.BlockSpec((1, _D), lambda i: (0, 0)),
    ],
    out_shape=[
        jax.ShapeDtypeStruct((_N, _D), jnp.float32),
        jax.ShapeDtypeStruct((1, _D), jnp.float32),
        jax.ShapeDtypeStruct((1, _D), jnp.float32),
    ],
)

_linear_plain = pl.pallas_call(
    _linear_plain_body,
    grid=(_N // _BN,),
    in_specs=_lin_in_specs,
    out_specs=pl.BlockSpec((_BN, _D), lambda i: (i, 0)),
    out_shape=jax.ShapeDtypeStruct((_N, _D), jnp.float32),
)


def _bn_relu_body(z_ref, sum_ref, sq_ref, g_ref, be_ref, o_ref):
    m = sum_ref[...] * (1.0 / _N)
    v = sq_ref[...] * (1.0 / _N) - m * m
    sc = g_ref[...] * lax.rsqrt(v + 1e-5)
    o_ref[...] = jnp.maximum((z_ref[...] - m) * sc + be_ref[...], 0.0)


_bn_relu = pl.pallas_call(
    _bn_relu_body,
    grid=(_N // _BN,),
    in_specs=[
        pl.BlockSpec((_BN, _D), lambda i: (i, 0)),
        pl.BlockSpec((1, _D), lambda i: (0, 0)),
        pl.BlockSpec((1, _D), lambda i: (0, 0)),
        pl.BlockSpec((1, _D), lambda i: (0, 0)),
        pl.BlockSpec((1, _D), lambda i: (0, 0)),
    ],
    out_specs=pl.BlockSpec((_BN, _D), lambda i: (i, 0)),
    out_shape=jax.ShapeDtypeStruct((_N, _D), jnp.float32),
)


def kernel(x, edge_index, edge_attr, Wl1, bl1, Wr1, br1, Wl2, bl2, Wr2, br2,
           Wl3, bl3, Wr3, br3, g1, be1, g2, be2):
    src = edge_index[0]
    dst = edge_index[1]
    cnt_parts = _count(dst).reshape(_NC, _N, _D)[:, :, :1]

    h = x
    layers = [
        (Wl1, bl1, Wr1, br1, g1, be1),
        (Wl2, bl2, Wr2, br2, g2, be2),
        (Wl3, bl3, Wr3, br3, None, None),
    ]
    z = None
    for li, (Wl, bl, Wr, br, g, be) in enumerate(layers):
        parts = _segsum(h, src, dst, edge_attr).reshape(_NC, _N, _D)
        bias = (bl + br).reshape(1, _D)
        if li < 2:
            z, ssum, ssq = _linear_stats(parts, cnt_parts, h, Wl.T, Wr.T, bias)
            h = _bn_relu(z, ssum, ssq, g.reshape(1, _D), be.reshape(1, _D))
        else:
            z = _linear_plain(parts, cnt_parts, h, Wl.T, Wr.T, bias)
    return z
